# Initial kernel scaffold; baseline (speedup 1.0000x reference)
import jax
import jax.numpy as jnp
from jax.experimental import pallas as pl


def kernel(x, edge_index, W1, b1, W2, b2, W3, b3):
    raise NotImplementedError("write your pallas kernel here")


# speedup vs baseline: 12.0844x; 12.0844x over previous
import functools

import jax
import jax.numpy as jnp
from jax import lax
from jax.experimental import pallas as pl
from jax.experimental.pallas import tpu as pltpu
from jax.experimental.pallas import tpu_sc as plsc

N_NODES = 10000
IN_CH = 128
HID_CH = 128
OUT_CH = 64
N_EDGES = 320000

NC = 2
NS = 16
NW = NC * NS
DEG_W = 16

_E_PER_TILE = N_EDGES // NW
_CHUNK = 80
_N_CHUNKS = _E_PER_TILE // _CHUNK
_ROWS_PER_TILE = N_NODES // NS
_STAGE_ROWS = 125
_N_STAGE = _ROWS_PER_TILE // _STAGE_ROWS


def _make_deg_kernel():
    mesh = plsc.VectorSubcoreMesh(core_axis_name="c", subcore_axis_name="s")

    @functools.partial(
        pl.kernel,
        out_type=jax.ShapeDtypeStruct((NC, N_NODES, DEG_W), jnp.float32),
        mesh=mesh,
        compiler_params=pltpu.CompilerParams(use_tc_tiling_on_sc=False),
        scratch_types=[
            pltpu.VMEM((_CHUNK,), jnp.int32),
            pltpu.VMEM((_CHUNK, DEG_W), jnp.float32),
            pltpu.VMEM((_ROWS_PER_TILE, DEG_W), jnp.float32),
            pltpu.VMEM_SHARED((N_NODES, DEG_W), jnp.float32),
        ],
    )
    def deg_kernel(dst_hbm, out_hbm, dst_v, ones_v, stage_v, acc):
        c = lax.axis_index("c")
        s = lax.axis_index("s")
        row0 = s * _ROWS_PER_TILE

        def fill_stage(i, _):
            stage_v[i, :] = jnp.zeros((16,), jnp.float32)
            return 0

        lax.fori_loop(0, _ROWS_PER_TILE, fill_stage, 0)

        def fill_ones(i, _):
            ones_v[i, :] = jnp.ones((16,), jnp.float32)
            return 0

        lax.fori_loop(0, _CHUNK, fill_ones, 0)

        pltpu.sync_copy(stage_v, acc.at[pl.ds(row0, _ROWS_PER_TILE)])
        plsc.subcore_barrier()

        def edge_step(i, _):
            off = c * (N_EDGES // NC) + s * _E_PER_TILE + i * _CHUNK
            pltpu.sync_copy(dst_hbm.at[pl.ds(off, _CHUNK)], dst_v)
            pltpu.sync_copy(ones_v, acc.at[dst_v], add=True)
            return 0

        lax.fori_loop(0, _N_CHUNKS, edge_step, 0)
        plsc.subcore_barrier()

        pltpu.sync_copy(acc.at[pl.ds(row0, _ROWS_PER_TILE)], stage_v)
        pltpu.sync_copy(stage_v, out_hbm.at[c, pl.ds(row0, _ROWS_PER_TILE)])

    return deg_kernel


def _make_agg_kernel(D):
    mesh = plsc.VectorSubcoreMesh(core_axis_name="c", subcore_axis_name="s")

    @functools.partial(
        pl.kernel,
        out_type=jax.ShapeDtypeStruct((NC, N_NODES, D), jnp.float32),
        mesh=mesh,
        compiler_params=pltpu.CompilerParams(use_tc_tiling_on_sc=False),
        scratch_types=[
            pltpu.VMEM((_CHUNK,), jnp.int32),
            pltpu.VMEM((_CHUNK,), jnp.int32),
            pltpu.VMEM((_CHUNK, D), jnp.float32),
            pltpu.VMEM((_STAGE_ROWS, D), jnp.float32),
            pltpu.VMEM_SHARED((N_NODES, D), jnp.float32),
            pltpu.SemaphoreType.DMA,
        ],
    )
    def agg_kernel(hs_hbm, src_hbm, dst_hbm, out_hbm,
                   src_v, dst_v, rows_v, stage_v, acc, sem):
        c = lax.axis_index("c")
        s = lax.axis_index("s")
        row0 = s * _ROWS_PER_TILE

        def fill_stage(i, _):
            for j in range(D // 16):
                stage_v[i, pl.ds(j * 16, 16)] = jnp.zeros((16,), jnp.float32)
            return 0

        lax.fori_loop(0, _STAGE_ROWS, fill_stage, 0)

        def zero_acc(k, _):
            pltpu.sync_copy(stage_v, acc.at[pl.ds(row0 + k * _STAGE_ROWS, _STAGE_ROWS)])
            return 0

        lax.fori_loop(0, _N_STAGE, zero_acc, 0)
        plsc.subcore_barrier()

        def edge_step(i, _):
            off = c * (N_EDGES // NC) + s * _E_PER_TILE + i * _CHUNK
            pltpu.sync_copy(src_hbm.at[pl.ds(off, _CHUNK)], src_v)
            pltpu.sync_copy(dst_hbm.at[pl.ds(off, _CHUNK)], dst_v)
            pltpu.async_copy(hs_hbm.at[src_v], rows_v, sem).wait()
            pltpu.sync_copy(rows_v, acc.at[dst_v], add=True)
            return 0

        lax.fori_loop(0, _N_CHUNKS, edge_step, 0)
        plsc.subcore_barrier()

        def writeout(k, _):
            pltpu.sync_copy(acc.at[pl.ds(row0 + k * _STAGE_ROWS, _STAGE_ROWS)], stage_v)
            pltpu.sync_copy(stage_v, out_hbm.at[c, pl.ds(row0 + k * _STAGE_ROWS, _STAGE_ROWS)])
            return 0

        lax.fori_loop(0, _N_STAGE, writeout, 0)

    return agg_kernel


_deg = _make_deg_kernel()
_agg128 = _make_agg_kernel(HID_CH)
_agg64 = _make_agg_kernel(OUT_CH)

_BLK = 2000
_GRID = N_NODES // _BLK


def _dinv_rows(dp_ref):
    deg = dp_ref[0] + dp_ref[1] + 1.0
    return jnp.min(lax.rsqrt(deg), axis=1, keepdims=True)


def _prep_body(dp_ref, x_ref, w_ref, o_ref):
    dinv = _dinv_rows(dp_ref)
    h = jnp.dot(x_ref[...], w_ref[...], preferred_element_type=jnp.float32)
    o_ref[...] = dinv * h


def _mid_body(dp_ref, p_ref, hs_ref, b_ref, w_ref, o_ref):
    dinv = _dinv_rows(dp_ref)
    z = dinv * (p_ref[0] + p_ref[1] + hs_ref[...]) + b_ref[...]
    z = jnp.maximum(z, 0.0)
    h = jnp.dot(z, w_ref[...], preferred_element_type=jnp.float32)
    o_ref[...] = dinv * h


def _final_body(dp_ref, p_ref, hs_ref, b_ref, o_ref):
    dinv = _dinv_rows(dp_ref)
    t = dinv * (p_ref[0] + p_ref[1] + hs_ref[...]) + b_ref[...]
    m = jnp.max(t, axis=1, keepdims=True)
    e = jnp.exp(t - m)
    lse = jnp.log(jnp.sum(e, axis=1, keepdims=True))
    o_ref[...] = t - m - lse


def _dp_spec():
    return pl.BlockSpec((NC, _BLK, DEG_W), lambda i: (0, i, 0))


def _rows_spec(D):
    return pl.BlockSpec((_BLK, D), lambda i: (i, 0))


def _part_spec(D):
    return pl.BlockSpec((NC, _BLK, D), lambda i: (0, i, 0))


def _vec_spec(D):
    return pl.BlockSpec((D,), lambda i: (0,))


def _mat_spec(Din, Dout):
    return pl.BlockSpec((Din, Dout), lambda i: (0, 0))


def _prep(dp, x, W1):
    return pl.pallas_call(
        _prep_body,
        grid=(_GRID,),
        in_specs=[_dp_spec(), _rows_spec(IN_CH), _mat_spec(IN_CH, HID_CH)],
        out_specs=_rows_spec(HID_CH),
        out_shape=jax.ShapeDtypeStruct((N_NODES, HID_CH), jnp.float32),
    )(dp, x, W1)


def _mid(dp, p, hs, b, W, Dout):
    D = hs.shape[1]
    return pl.pallas_call(
        _mid_body,
        grid=(_GRID,),
        in_specs=[_dp_spec(), _part_spec(D), _rows_spec(D), _vec_spec(D),
                  _mat_spec(D, Dout)],
        out_specs=_rows_spec(Dout),
        out_shape=jax.ShapeDtypeStruct((N_NODES, Dout), jnp.float32),
    )(dp, p, hs, b, W)


def _final(dp, p, hs, b):
    return pl.pallas_call(
        _final_body,
        grid=(_GRID,),
        in_specs=[_dp_spec(), _part_spec(OUT_CH), _rows_spec(OUT_CH),
                  _vec_spec(OUT_CH)],
        out_specs=_rows_spec(OUT_CH),
        out_shape=jax.ShapeDtypeStruct((N_NODES, OUT_CH), jnp.float32),
    )(dp, p, hs, b)


def kernel(x, edge_index, W1, b1, W2, b2, W3, b3):
    src = edge_index[0].astype(jnp.int32)
    dst = edge_index[1].astype(jnp.int32)
    dp = _deg(dst)
    hs1 = _prep(dp, x, W1)
    p1 = _agg128(hs1, src, dst)
    hs2 = _mid(dp, p1, hs1, b1, W2, HID_CH)
    p2 = _agg128(hs2, src, dst)
    hs3 = _mid(dp, p2, hs2, b2, W3, OUT_CH)
    p3 = _agg64(hs3, src, dst)
    return _final(dp, p3, hs3, b3)


# speedup vs baseline: 24.4257x; 2.0213x over previous
import functools

import jax
import jax.numpy as jnp
from jax import lax
from jax.experimental import pallas as pl
from jax.experimental.pallas import tpu as pltpu
from jax.experimental.pallas import tpu_sc as plsc

N_NODES = 10000
IN_CH = 128
HID_CH = 128
OUT_CH = 64
N_EDGES = 320000

NC = 2
NS = 16
NW = NC * NS
DEG_W = 16

_E_PER_TILE = N_EDGES // NW
_ROWS_PER_TILE = N_NODES // NS

_CHUNK = 50
_N_CHUNKS = _E_PER_TILE // _CHUNK
_K = 2
_NSUP = _N_CHUNKS // _K
_NPAIR = _NSUP // 2
_WFULL = _ROWS_PER_TILE // _CHUNK
_WREM = _ROWS_PER_TILE - _WFULL * _CHUNK

_DCHUNK = 125
_DN_CHUNKS = _E_PER_TILE // _DCHUNK
_DWIN = 8


def _make_deg_kernel():
    mesh = plsc.VectorSubcoreMesh(core_axis_name="c", subcore_axis_name="s")

    @functools.partial(
        pl.kernel,
        out_type=jax.ShapeDtypeStruct((NC, N_NODES, DEG_W), jnp.float32),
        mesh=mesh,
        compiler_params=pltpu.CompilerParams(use_tc_tiling_on_sc=False),
        scratch_types=[
            pltpu.VMEM((_DN_CHUNKS, _DCHUNK), jnp.int32),
            pltpu.VMEM((_DCHUNK, DEG_W), jnp.float32),
            pltpu.VMEM((_ROWS_PER_TILE, DEG_W), jnp.float32),
            pltpu.VMEM_SHARED((N_NODES, DEG_W), jnp.float32),
            pltpu.SemaphoreType.DMA,
        ],
    )
    def deg_kernel(dst_hbm, out_hbm, dst_t, ones_v, stage_v, acc, ssem):
        c = lax.axis_index("c")
        s = lax.axis_index("s")
        tid = c * NS + s
        row0 = s * _ROWS_PER_TILE

        def fill_stage(i, _):
            stage_v[i, :] = jnp.zeros((16,), jnp.float32)
            return 0

        lax.fori_loop(0, _ROWS_PER_TILE, fill_stage, 0)

        def fill_ones(i, _):
            ones_v[i, :] = jnp.ones((16,), jnp.float32)
            return 0

        lax.fori_loop(0, _DCHUNK, fill_ones, 0)

        pltpu.sync_copy(dst_hbm.at[tid], dst_t)
        pltpu.sync_copy(stage_v, acc.at[pl.ds(row0, _ROWS_PER_TILE)])
        plsc.subcore_barrier()

        def drain_one():
            pltpu.make_async_copy(
                ones_v, acc.at[pl.ds(0, _DCHUNK)], ssem).wait()

        def edge_step(i, _):
            pltpu.async_copy(ones_v, acc.at[dst_t.at[i]], ssem, add=True)

            @pl.when(i >= _DWIN)
            def _():
                drain_one()

            return 0

        lax.fori_loop(0, _DN_CHUNKS, edge_step, 0)
        for _ in range(_DWIN):
            drain_one()
        plsc.subcore_barrier()

        pltpu.sync_copy(acc.at[pl.ds(row0, _ROWS_PER_TILE)], stage_v)
        pltpu.sync_copy(stage_v, out_hbm.at[c, pl.ds(row0, _ROWS_PER_TILE)])

    return deg_kernel


def _make_agg_kernel(D):
    mesh = plsc.VectorSubcoreMesh(core_axis_name="c", subcore_axis_name="s")

    @functools.partial(
        pl.kernel,
        out_type=jax.ShapeDtypeStruct((NC, N_NODES, D), jnp.float32),
        mesh=mesh,
        compiler_params=pltpu.CompilerParams(use_tc_tiling_on_sc=False),
        scratch_types=[
            pltpu.VMEM((_N_CHUNKS, _CHUNK), jnp.int32),
            pltpu.VMEM((_N_CHUNKS, _CHUNK), jnp.int32),
            pltpu.VMEM((2, _K, _CHUNK, D), jnp.float32),
            pltpu.VMEM_SHARED((N_NODES, D), jnp.float32),
            pltpu.SemaphoreType.DMA,
            pltpu.SemaphoreType.DMA,
        ],
    )
    def agg_kernel(hs_hbm, src_hbm, dst_hbm, out_hbm,
                   src_t, dst_t, rows, acc, gsem, ssem):
        c = lax.axis_index("c")
        s = lax.axis_index("s")
        tid = c * NS + s
        row0 = s * _ROWS_PER_TILE
        stage_v = rows.at[0, 0]

        pltpu.sync_copy(src_hbm.at[tid], src_t)
        pltpu.sync_copy(dst_hbm.at[tid], dst_t)

        def fill_stage(i, _):
            for j in range(D // 16):
                rows[0, 0, i, pl.ds(j * 16, 16)] = jnp.zeros((16,), jnp.float32)
            return 0

        lax.fori_loop(0, _CHUNK, fill_stage, 0)

        def zero_acc(k, _):
            pltpu.sync_copy(
                stage_v, acc.at[pl.ds(row0 + k * _CHUNK, _CHUNK)])
            return 0

        lax.fori_loop(0, _WFULL, zero_acc, 0)
        pltpu.sync_copy(
            rows.at[0, 0, pl.ds(0, _WREM)],
            acc.at[pl.ds(row0 + _WFULL * _CHUNK, _WREM)])
        plsc.subcore_barrier()

        def issue_gather(g, k, chunk):
            pltpu.async_copy(hs_hbm.at[src_t.at[chunk]], rows.at[g, k], gsem)

        def wait_gather(g, k):
            pltpu.make_async_copy(
                hs_hbm.at[pl.ds(0, _CHUNK)], rows.at[g, k], gsem).wait()

        def issue_scatter(g, k, chunk):
            pltpu.async_copy(rows.at[g, k], acc.at[dst_t.at[chunk]], ssem,
                             add=True)

        def drain_scatter(g, k):
            pltpu.make_async_copy(
                rows.at[g, k], acc.at[pl.ds(0, _CHUNK)], ssem).wait()

        for k in range(_K):
            issue_gather(0, k, k)

        def pair_step(j2, _):
            base0 = (2 * j2) * _K
            base1 = base0 + _K
            for k in range(_K):
                wait_gather(0, k)

            @pl.when(j2 > 0)
            def _():
                for k in range(_K):
                    drain_scatter(1, k)

            for k in range(_K):
                issue_gather(1, k, base1 + k)
            for k in range(_K):
                issue_scatter(0, k, base0 + k)
            for k in range(_K):
                wait_gather(1, k)
            for k in range(_K):
                drain_scatter(0, k)

            @pl.when(j2 < _NPAIR - 1)
            def _():
                for k in range(_K):
                    issue_gather(0, k, base1 + _K + k)

            for k in range(_K):
                issue_scatter(1, k, base1 + k)
            return 0

        lax.fori_loop(0, _NPAIR, pair_step, 0)
        for k in range(_K):
            drain_scatter(1, k)
        plsc.subcore_barrier()

        def writeout(k, _):
            pltpu.sync_copy(
                acc.at[pl.ds(row0 + k * _CHUNK, _CHUNK)], stage_v)
            pltpu.sync_copy(
                stage_v, out_hbm.at[c, pl.ds(row0 + k * _CHUNK, _CHUNK)])
            return 0

        lax.fori_loop(0, _WFULL, writeout, 0)
        pltpu.sync_copy(
            acc.at[pl.ds(row0 + _WFULL * _CHUNK, _WREM)],
            rows.at[0, 0, pl.ds(0, _WREM)])
        pltpu.sync_copy(
            rows.at[0, 0, pl.ds(0, _WREM)],
            out_hbm.at[c, pl.ds(row0 + _WFULL * _CHUNK, _WREM)])

    return agg_kernel


_deg = _make_deg_kernel()
_agg128 = _make_agg_kernel(HID_CH)
_agg64 = _make_agg_kernel(OUT_CH)

_BLK = 2000
_GRID = N_NODES // _BLK


def _dinv_rows(dp_ref):
    deg = dp_ref[0] + dp_ref[1] + 1.0
    return jnp.min(lax.rsqrt(deg), axis=1, keepdims=True)


def _prep_body(dp_ref, x_ref, w_ref, o_ref):
    dinv = _dinv_rows(dp_ref)
    h = jnp.dot(x_ref[...], w_ref[...], preferred_element_type=jnp.float32)
    o_ref[...] = dinv * h


def _mid_body(dp_ref, p_ref, hs_ref, b_ref, w_ref, o_ref):
    dinv = _dinv_rows(dp_ref)
    z = dinv * (p_ref[0] + p_ref[1] + hs_ref[...]) + b_ref[...]
    z = jnp.maximum(z, 0.0)
    h = jnp.dot(z, w_ref[...], preferred_element_type=jnp.float32)
    o_ref[...] = dinv * h


def _final_body(dp_ref, p_ref, hs_ref, b_ref, o_ref):
    dinv = _dinv_rows(dp_ref)
    t = dinv * (p_ref[0] + p_ref[1] + hs_ref[...]) + b_ref[...]
    m = jnp.max(t, axis=1, keepdims=True)
    e = jnp.exp(t - m)
    lse = jnp.log(jnp.sum(e, axis=1, keepdims=True))
    o_ref[...] = t - m - lse


def _dp_spec():
    return pl.BlockSpec((NC, _BLK, DEG_W), lambda i: (0, i, 0))


def _rows_spec(D):
    return pl.BlockSpec((_BLK, D), lambda i: (i, 0))


def _part_spec(D):
    return pl.BlockSpec((NC, _BLK, D), lambda i: (0, i, 0))


def _vec_spec(D):
    return pl.BlockSpec((D,), lambda i: (0,))


def _mat_spec(Din, Dout):
    return pl.BlockSpec((Din, Dout), lambda i: (0, 0))


def _prep(dp, x, W1):
    return pl.pallas_call(
        _prep_body,
        grid=(_GRID,),
        in_specs=[_dp_spec(), _rows_spec(IN_CH), _mat_spec(IN_CH, HID_CH)],
        out_specs=_rows_spec(HID_CH),
        out_shape=jax.ShapeDtypeStruct((N_NODES, HID_CH), jnp.float32),
    )(dp, x, W1)


def _mid(dp, p, hs, b, W, Dout):
    D = hs.shape[1]
    return pl.pallas_call(
        _mid_body,
        grid=(_GRID,),
        in_specs=[_dp_spec(), _part_spec(D), _rows_spec(D), _vec_spec(D),
                  _mat_spec(D, Dout)],
        out_specs=_rows_spec(Dout),
        out_shape=jax.ShapeDtypeStruct((N_NODES, Dout), jnp.float32),
    )(dp, p, hs, b, W)


def _final(dp, p, hs, b):
    return pl.pallas_call(
        _final_body,
        grid=(_GRID,),
        in_specs=[_dp_spec(), _part_spec(OUT_CH), _rows_spec(OUT_CH),
                  _vec_spec(OUT_CH)],
        out_specs=_rows_spec(OUT_CH),
        out_shape=jax.ShapeDtypeStruct((N_NODES, OUT_CH), jnp.float32),
    )(dp, p, hs, b)


def kernel(x, edge_index, W1, b1, W2, b2, W3, b3):
    src = edge_index[0].astype(jnp.int32)
    dst = edge_index[1].astype(jnp.int32)
    src3 = src.reshape(NW, _N_CHUNKS, _CHUNK)
    dst3 = dst.reshape(NW, _N_CHUNKS, _CHUNK)
    dstd = dst.reshape(NW, _DN_CHUNKS, _DCHUNK)
    dp = _deg(dstd)
    hs1 = _prep(dp, x, W1)
    p1 = _agg128(hs1, src3, dst3)
    hs2 = _mid(dp, p1, hs1, b1, W2, HID_CH)
    p2 = _agg128(hs2, src3, dst3)
    hs3 = _mid(dp, p2, hs2, b2, W3, OUT_CH)
    p3 = _agg64(hs3, src3, dst3)
    return _final(dp, p3, hs3, b3)
